# baseline (speedup 1.0000x reference)
import jax
import jax.numpy as jnp
from jax import lax
from jax.experimental import pallas as pl
from jax.experimental.pallas import tpu as pltpu
from jax.experimental.pallas import tpu_sc as plsc

B = 8
C = 32
N = 100000
R = 32
R3 = R * R * R
L = 16

CH1 = 2000
G1 = CH1 // L
CH2 = 10000
G2 = CH2 // L
NCH2 = N // CH2
U = 5
SUB = 8192

CLIP_HI = 1.0 - 1e-6
RANGE = 2.2


def _voxel_index(xv, yv, zv):
    def quant(v):
        c = jnp.clip(v / jnp.float32(RANGE) + jnp.float32(0.5),
                     jnp.float32(0.0), jnp.float32(CLIP_HI))
        g = (c * jnp.float32(R)).astype(jnp.int32)
        return jnp.clip(g, 0, R - 1)
    return quant(xv) + R * quant(yv) + (R * R) * quant(zv)


def _body(pts_ref, feat_ref, out_ref, idx_ref, pc_ref, rc_ref,
          bins0, bins1, ibA, ibB, fA0, fA1, fB0, fB1, semA, semB):
    cid = lax.axis_index("c")
    sid = lax.axis_index("s")
    w = cid * 16 + sid
    b = w // 4
    q = w % 4

    zeros = jnp.zeros((L,), jnp.float32)
    ones = jnp.ones((L,), jnp.float32)
    stride3 = jnp.arange(L, dtype=jnp.int32) * 3

    def zero_body(i, _):
        o = pl.multiple_of(i * (4 * L), L)
        for u in range(4):
            bins0[pl.ds(o + u * L, L)] = zeros
        return 0
    lax.fori_loop(0, R3 // (4 * L), zero_body, 0)

    nch = jnp.where(q < 2, 13, 12)
    ch_start = q * 13 - jnp.maximum(q - 2, 0)

    def p1_chunk(i, _):
        n0 = (ch_start + i) * CH1
        pbase = b * (3 * N) + 3 * n0
        pltpu.sync_copy(pts_ref.at[pl.ds(pl.multiple_of(pbase, 8), 3 * CH1)],
                        fA0.at[pl.ds(0, 3 * CH1)])

        def p1_group(g, _):
            base = g * (3 * L)
            ix = stride3 + base
            iv = _voxel_index(plsc.load_gather(fA0, [ix]),
                              plsc.load_gather(fA0, [ix + 1]),
                              plsc.load_gather(fA0, [ix + 2]))
            ibA[pl.ds(pl.multiple_of(g * L, L), L)] = iv
            plsc.addupdate_scatter(bins0, [iv], ones)
            return 0
        lax.fori_loop(0, G1, p1_group, 0)

        dst = b * N + n0
        pltpu.sync_copy(ibA.at[pl.ds(0, CH1)],
                        idx_ref.at[pl.ds(pl.multiple_of(dst, 8), CH1)])
        return 0
    lax.fori_loop(0, nch, p1_chunk, 0)

    pltpu.sync_copy(bins0.at[pl.ds(0, R3)],
                    pc_ref.at[pl.ds(pl.multiple_of(w * R3, 8), R3)])

    plsc.subcore_barrier()

    PS = 2048
    for sub in range(4):
        off = q * (R3 // 4) + sub * PS
        base = 4 * b * R3 + off
        pltpu.sync_copy(pc_ref.at[pl.ds(pl.multiple_of(base, 8), PS)],
                        fA0.at[pl.ds(0, PS)])
        pltpu.sync_copy(pc_ref.at[pl.ds(pl.multiple_of(base + R3, 8), PS)],
                        fA1.at[pl.ds(0, PS)])
        pltpu.sync_copy(pc_ref.at[pl.ds(pl.multiple_of(base + 2 * R3, 8), PS)],
                        fB0.at[pl.ds(0, PS)])
        pltpu.sync_copy(pc_ref.at[pl.ds(pl.multiple_of(base + 3 * R3, 8), PS)],
                        fB1.at[pl.ds(0, PS)])

        def rc_group(g, _):
            o = pl.multiple_of(g * L, L)
            s = (fA0[pl.ds(o, L)] + fA1[pl.ds(o, L)]
                 + fB0[pl.ds(o, L)] + fB1[pl.ds(o, L)])
            fA0[pl.ds(o, L)] = jnp.float32(1.0) / jnp.maximum(s, jnp.float32(1.0))
            return 0
        lax.fori_loop(0, PS // L, rc_group, 0)

        pltpu.sync_copy(fA0.at[pl.ds(0, PS)],
                        rc_ref.at[pl.ds(pl.multiple_of(b * R3 + off, 8), PS)])

    plsc.subcore_barrier()

    bufs = ((ibA, fA0, fA1, semA), (ibB, fB0, fB1, semB))

    def issue(ch, c0):
        ib, f0, f1, sem = bufs[ch % 2]
        n0 = ch * CH2
        h0 = pltpu.async_copy(
            idx_ref.at[pl.ds(pl.multiple_of(b * N + n0, 8), CH2)],
            ib.at[pl.ds(0, CH2)], sem)
        h1 = pltpu.async_copy(
            feat_ref.at[pl.ds(pl.multiple_of((b * C + c0) * N + n0, 8), CH2)],
            f0.at[pl.ds(0, CH2)], sem)
        h2 = pltpu.async_copy(
            feat_ref.at[pl.ds(pl.multiple_of((b * C + c0 + 1) * N + n0, 8), CH2)],
            f1.at[pl.ds(0, CH2)], sem)
        return (h0, h1, h2)

    out_handles = []
    for p in range(4):
        c0 = p * 8 + q * 2

        pending = issue(0, c0)

        for h in out_handles:
            h.wait()
        out_handles = []

        def zero2(i, _):
            o = pl.multiple_of(i * (4 * L), L)
            for u in range(4):
                bins0[pl.ds(o + u * L, L)] = zeros
                bins1[pl.ds(o + u * L, L)] = zeros
            return 0
        lax.fori_loop(0, R3 // (4 * L), zero2, 0)

        for ch in range(NCH2):
            nxt = issue(ch + 1, c0) if ch + 1 < NCH2 else None
            for h in pending:
                h.wait()
            ib, f0, f1, _ = bufs[ch % 2]

            def p2_group(i, _):
                o = pl.multiple_of(i * (U * L), L)
                for u in range(U):
                    ou = o + u * L
                    iv = ib[pl.ds(ou, L)]
                    plsc.addupdate_scatter(bins0, [iv], f0[pl.ds(ou, L)])
                    plsc.addupdate_scatter(bins1, [iv], f1[pl.ds(ou, L)])
                return 0
            lax.fori_loop(0, G2 // U, p2_group, 0)
            pending = nxt

        for sl in range(R3 // SUB):
            off = sl * SUB
            pltpu.sync_copy(
                rc_ref.at[pl.ds(pl.multiple_of(b * R3 + off, 8), SUB)],
                fA0.at[pl.ds(0, SUB)])

            def scale(g, _):
                o = pl.multiple_of(g * (4 * L), L)
                for u in range(4):
                    ou = o + u * L
                    r = fA0[pl.ds(ou, L)]
                    bins0[pl.ds(off + ou, L)] = bins0[pl.ds(off + ou, L)] * r
                    bins1[pl.ds(off + ou, L)] = bins1[pl.ds(off + ou, L)] * r
                return 0
            lax.fori_loop(0, SUB // (4 * L), scale, 0)

            obase = (b * C + c0) * R3 + off
            out_handles.append(pltpu.async_copy(
                bins0.at[pl.ds(off, SUB)],
                out_ref.at[pl.ds(pl.multiple_of(obase, 8), SUB)], semB))
            out_handles.append(pltpu.async_copy(
                bins1.at[pl.ds(off, SUB)],
                out_ref.at[pl.ds(pl.multiple_of(obase + R3, 8), SUB)], semB))

    for h in out_handles:
        h.wait()


@jax.jit
def _grid_encode(pts_flat, feat_flat):
    mesh = plsc.VectorSubcoreMesh(core_axis_name="c", subcore_axis_name="s")
    fn = pl.kernel(
        _body,
        out_type=(
            jax.ShapeDtypeStruct((B * C * R3,), jnp.float32),
            jax.ShapeDtypeStruct((B * N,), jnp.int32),
            jax.ShapeDtypeStruct((32 * R3,), jnp.float32),
            jax.ShapeDtypeStruct((B * R3,), jnp.float32),
        ),
        mesh=mesh,
        compiler_params=pltpu.CompilerParams(needs_layout_passes=False),
        scratch_types=[
            pltpu.VMEM((R3,), jnp.float32),
            pltpu.VMEM((R3,), jnp.float32),
            pltpu.VMEM((CH2,), jnp.int32),
            pltpu.VMEM((CH2,), jnp.int32),
            pltpu.VMEM((CH2,), jnp.float32),
            pltpu.VMEM((CH2,), jnp.float32),
            pltpu.VMEM((CH2,), jnp.float32),
            pltpu.VMEM((CH2,), jnp.float32),
            pltpu.SemaphoreType.DMA,
            pltpu.SemaphoreType.DMA,
        ],
    )
    return fn(pts_flat, feat_flat)


def kernel(points, feature):
    pts_flat = points.reshape(-1)
    feat_flat = feature.reshape(-1)
    grid, _, _, _ = _grid_encode(pts_flat, feat_flat)
    return grid.reshape(B, C, R, R, R)


# speedup vs baseline: 4.5732x; 4.5732x over previous
import jax
import jax.numpy as jnp
from jax import lax
from jax.experimental import pallas as pl
from jax.experimental.pallas import tpu as pltpu
from jax.experimental.pallas import tpu_sc as plsc

B = 8
C = 32
N = 100000
R = 32
R3 = R * R * R
L = 16

CH1 = 2000
G1 = CH1 // L
CH2 = 10000
G2 = CH2 // L
NCH2 = N // CH2
U = 5
SUB = 8192

CLIP_HI = 1.0 - 1e-6
RANGE = 2.2


def _voxel_index(xv, yv, zv):
    def quant(v):
        c = jnp.clip(v / jnp.float32(RANGE) + jnp.float32(0.5),
                     jnp.float32(0.0), jnp.float32(CLIP_HI))
        g = (c * jnp.float32(R)).astype(jnp.int32)
        return jnp.clip(g, 0, R - 1)
    return quant(xv) + R * quant(yv) + (R * R) * quant(zv)


def _body(pts_ref, feat_ref, out_ref, idx_ref, pc_ref, rc_ref,
          bins0, bins1, ibA, ibB, fA0, fA1, fB0, fB1, semA, semB):
    cid = lax.axis_index("c")
    sid = lax.axis_index("s")
    w = cid * 16 + sid
    b = w // 4
    q = w % 4

    zeros = jnp.zeros((L,), jnp.float32)
    ones = jnp.ones((L,), jnp.float32)

    def zero_body(i, _):
        o = pl.multiple_of(i * (4 * L), L)
        for u in range(4):
            bins0[pl.ds(o + u * L, L)] = zeros
        return 0
    lax.fori_loop(0, R3 // (4 * L), zero_body, 0)

    nch = jnp.where(q < 2, 13, 12)
    ch_start = q * 13 - jnp.maximum(q - 2, 0)

    def p1_chunk(i, _):
        n0 = (ch_start + i) * CH1
        pbase = b * (3 * N) + n0
        pltpu.sync_copy(pts_ref.at[pl.ds(pl.multiple_of(pbase, 8), CH1)],
                        fA0.at[pl.ds(0, CH1)])
        pltpu.sync_copy(pts_ref.at[pl.ds(pl.multiple_of(pbase + N, 8), CH1)],
                        fA1.at[pl.ds(0, CH1)])
        pltpu.sync_copy(pts_ref.at[pl.ds(pl.multiple_of(pbase + 2 * N, 8), CH1)],
                        fB0.at[pl.ds(0, CH1)])

        def p1_group(g, _):
            o = pl.multiple_of(g * L, L)
            iv = _voxel_index(fA0[pl.ds(o, L)], fA1[pl.ds(o, L)],
                              fB0[pl.ds(o, L)])
            ibA[pl.ds(pl.multiple_of(g * L, L), L)] = iv
            plsc.addupdate_scatter(bins0, [iv], ones)
            return 0
        lax.fori_loop(0, G1, p1_group, 0)

        dst = b * N + n0
        pltpu.sync_copy(ibA.at[pl.ds(0, CH1)],
                        idx_ref.at[pl.ds(pl.multiple_of(dst, 8), CH1)])
        return 0
    lax.fori_loop(0, nch, p1_chunk, 0)

    pltpu.sync_copy(bins0.at[pl.ds(0, R3)],
                    pc_ref.at[pl.ds(pl.multiple_of(w * R3, 8), R3)])

    plsc.subcore_barrier()

    PS = 2048
    for sub in range(4):
        off = q * (R3 // 4) + sub * PS
        base = 4 * b * R3 + off
        pltpu.sync_copy(pc_ref.at[pl.ds(pl.multiple_of(base, 8), PS)],
                        fA0.at[pl.ds(0, PS)])
        pltpu.sync_copy(pc_ref.at[pl.ds(pl.multiple_of(base + R3, 8), PS)],
                        fA1.at[pl.ds(0, PS)])
        pltpu.sync_copy(pc_ref.at[pl.ds(pl.multiple_of(base + 2 * R3, 8), PS)],
                        fB0.at[pl.ds(0, PS)])
        pltpu.sync_copy(pc_ref.at[pl.ds(pl.multiple_of(base + 3 * R3, 8), PS)],
                        fB1.at[pl.ds(0, PS)])

        def rc_group(g, _):
            o = pl.multiple_of(g * L, L)
            s = (fA0[pl.ds(o, L)] + fA1[pl.ds(o, L)]
                 + fB0[pl.ds(o, L)] + fB1[pl.ds(o, L)])
            fA0[pl.ds(o, L)] = jnp.float32(1.0) / jnp.maximum(s, jnp.float32(1.0))
            return 0
        lax.fori_loop(0, PS // L, rc_group, 0)

        pltpu.sync_copy(fA0.at[pl.ds(0, PS)],
                        rc_ref.at[pl.ds(pl.multiple_of(b * R3 + off, 8), PS)])

    plsc.subcore_barrier()

    bufs = ((ibA, fA0, fA1, semA), (ibB, fB0, fB1, semB))

    def issue(ch, c0):
        ib, f0, f1, sem = bufs[ch % 2]
        n0 = ch * CH2
        h0 = pltpu.async_copy(
            idx_ref.at[pl.ds(pl.multiple_of(b * N + n0, 8), CH2)],
            ib.at[pl.ds(0, CH2)], sem)
        h1 = pltpu.async_copy(
            feat_ref.at[pl.ds(pl.multiple_of((b * C + c0) * N + n0, 8), CH2)],
            f0.at[pl.ds(0, CH2)], sem)
        h2 = pltpu.async_copy(
            feat_ref.at[pl.ds(pl.multiple_of((b * C + c0 + 1) * N + n0, 8), CH2)],
            f1.at[pl.ds(0, CH2)], sem)
        return (h0, h1, h2)

    out_handles = []
    for p in range(4):
        c0 = p * 8 + q * 2

        pending = issue(0, c0)

        for h in out_handles:
            h.wait()
        out_handles = []

        def zero2(i, _):
            o = pl.multiple_of(i * (4 * L), L)
            for u in range(4):
                bins0[pl.ds(o + u * L, L)] = zeros
                bins1[pl.ds(o + u * L, L)] = zeros
            return 0
        lax.fori_loop(0, R3 // (4 * L), zero2, 0)

        for ch in range(NCH2):
            nxt = issue(ch + 1, c0) if ch + 1 < NCH2 else None
            for h in pending:
                h.wait()
            ib, f0, f1, _ = bufs[ch % 2]

            def p2_group(i, _):
                o = pl.multiple_of(i * (U * L), L)
                for u in range(U):
                    ou = o + u * L
                    iv = ib[pl.ds(ou, L)]
                    plsc.addupdate_scatter(bins0, [iv], f0[pl.ds(ou, L)])
                    plsc.addupdate_scatter(bins1, [iv], f1[pl.ds(ou, L)])
                return 0
            lax.fori_loop(0, G2 // U, p2_group, 0)
            pending = nxt

        for sl in range(R3 // SUB):
            off = sl * SUB
            pltpu.sync_copy(
                rc_ref.at[pl.ds(pl.multiple_of(b * R3 + off, 8), SUB)],
                fA0.at[pl.ds(0, SUB)])

            def scale(g, _):
                o = pl.multiple_of(g * (4 * L), L)
                for u in range(4):
                    ou = o + u * L
                    r = fA0[pl.ds(ou, L)]
                    bins0[pl.ds(off + ou, L)] = bins0[pl.ds(off + ou, L)] * r
                    bins1[pl.ds(off + ou, L)] = bins1[pl.ds(off + ou, L)] * r
                return 0
            lax.fori_loop(0, SUB // (4 * L), scale, 0)

            obase = (b * C + c0) * R3 + off
            out_handles.append(pltpu.async_copy(
                bins0.at[pl.ds(off, SUB)],
                out_ref.at[pl.ds(pl.multiple_of(obase, 8), SUB)], semB))
            out_handles.append(pltpu.async_copy(
                bins1.at[pl.ds(off, SUB)],
                out_ref.at[pl.ds(pl.multiple_of(obase + R3, 8), SUB)], semB))

    for h in out_handles:
        h.wait()


@jax.jit
def _grid_encode(pts_flat, feat_flat):
    mesh = plsc.VectorSubcoreMesh(core_axis_name="c", subcore_axis_name="s")
    fn = pl.kernel(
        _body,
        out_type=(
            jax.ShapeDtypeStruct((B * C * R3,), jnp.float32),
            jax.ShapeDtypeStruct((B * N,), jnp.int32),
            jax.ShapeDtypeStruct((32 * R3,), jnp.float32),
            jax.ShapeDtypeStruct((B * R3,), jnp.float32),
        ),
        mesh=mesh,
        compiler_params=pltpu.CompilerParams(needs_layout_passes=False),
        scratch_types=[
            pltpu.VMEM((R3,), jnp.float32),
            pltpu.VMEM((R3,), jnp.float32),
            pltpu.VMEM((CH2,), jnp.int32),
            pltpu.VMEM((CH2,), jnp.int32),
            pltpu.VMEM((CH2,), jnp.float32),
            pltpu.VMEM((CH2,), jnp.float32),
            pltpu.VMEM((CH2,), jnp.float32),
            pltpu.VMEM((CH2,), jnp.float32),
            pltpu.SemaphoreType.DMA,
            pltpu.SemaphoreType.DMA,
        ],
    )
    return fn(pts_flat, feat_flat)


def kernel(points, feature):
    pts_flat = points.transpose(0, 2, 1).reshape(-1)
    feat_flat = feature.reshape(-1)
    grid, _, _, _ = _grid_encode(pts_flat, feat_flat)
    return grid.reshape(B, C, R, R, R)


# speedup vs baseline: 6.1140x; 1.3369x over previous
import jax
import jax.numpy as jnp
from jax import lax
from jax.experimental import pallas as pl
from jax.experimental.pallas import tpu as pltpu
from jax.experimental.pallas import tpu_sc as plsc

B = 8
C = 32
N = 100000
R = 32
R3 = R * R * R
L = 16

CH1 = 2000
G1 = CH1 // L
CH2 = 10000
G2 = CH2 // L
NCH2 = N // CH2
U = 5
SUB = 8192

CLIP_HI = 1.0 - 1e-6
RANGE = 2.2


def _voxel_index(xv, yv, zv):
    def quant(v):
        c = jnp.clip(v / jnp.float32(RANGE) + jnp.float32(0.5),
                     jnp.float32(0.0), jnp.float32(CLIP_HI))
        g = (c * jnp.float32(R)).astype(jnp.int32)
        return jnp.clip(g, 0, R - 1)
    return quant(xv) + R * quant(yv) + (R * R) * quant(zv)


def _body(pts_ref, feat_ref, out_ref, idx_ref, pc_ref, rc_ref,
          bins0, bins1, ibA, ibB, fA0, fA1, fB0, fB1, semA, semB):
    cid = lax.axis_index("c")
    sid = lax.axis_index("s")
    w = cid * 16 + sid
    b = w // 4
    q = w % 4

    zeros = jnp.zeros((L,), jnp.float32)
    ones = jnp.ones((L,), jnp.float32)

    @plsc.parallel_loop(0, R3 // L, 1, unroll=8)
    def zero_body(i):
        bins0[pl.ds(pl.multiple_of(i * L, L), L)] = zeros

    nch = jnp.where(q < 2, 13, 12)
    ch_start = q * 13 - jnp.maximum(q - 2, 0)

    def p1_chunk(i, _):
        n0 = (ch_start + i) * CH1
        pbase = b * (3 * N) + n0
        pltpu.sync_copy(pts_ref.at[pl.ds(pl.multiple_of(pbase, 8), CH1)],
                        fA0.at[pl.ds(0, CH1)])
        pltpu.sync_copy(pts_ref.at[pl.ds(pl.multiple_of(pbase + N, 8), CH1)],
                        fA1.at[pl.ds(0, CH1)])
        pltpu.sync_copy(pts_ref.at[pl.ds(pl.multiple_of(pbase + 2 * N, 8), CH1)],
                        fB0.at[pl.ds(0, CH1)])

        @plsc.parallel_loop(0, G1, 1, unroll=4)
        def p1_group(g):
            o = pl.multiple_of(g * L, L)
            iv = _voxel_index(fA0[pl.ds(o, L)], fA1[pl.ds(o, L)],
                              fB0[pl.ds(o, L)])
            ibA[pl.ds(o, L)] = iv
            plsc.addupdate_scatter(bins0, [iv], ones)

        dst = b * N + n0
        pltpu.sync_copy(ibA.at[pl.ds(0, CH1)],
                        idx_ref.at[pl.ds(pl.multiple_of(dst, 8), CH1)])
        return 0
    lax.fori_loop(0, nch, p1_chunk, 0)

    pltpu.sync_copy(bins0.at[pl.ds(0, R3)],
                    pc_ref.at[pl.ds(pl.multiple_of(w * R3, 8), R3)])

    plsc.subcore_barrier()

    PS = 2048
    for sub in range(4):
        off = q * (R3 // 4) + sub * PS
        base = 4 * b * R3 + off
        pltpu.sync_copy(pc_ref.at[pl.ds(pl.multiple_of(base, 8), PS)],
                        fA0.at[pl.ds(0, PS)])
        pltpu.sync_copy(pc_ref.at[pl.ds(pl.multiple_of(base + R3, 8), PS)],
                        fA1.at[pl.ds(0, PS)])
        pltpu.sync_copy(pc_ref.at[pl.ds(pl.multiple_of(base + 2 * R3, 8), PS)],
                        fB0.at[pl.ds(0, PS)])
        pltpu.sync_copy(pc_ref.at[pl.ds(pl.multiple_of(base + 3 * R3, 8), PS)],
                        fB1.at[pl.ds(0, PS)])

        @plsc.parallel_loop(0, PS // L, 1, unroll=4)
        def rc_group(g):
            o = pl.multiple_of(g * L, L)
            s = (fA0[pl.ds(o, L)] + fA1[pl.ds(o, L)]
                 + fB0[pl.ds(o, L)] + fB1[pl.ds(o, L)])
            fA0[pl.ds(o, L)] = jnp.float32(1.0) / jnp.maximum(s, jnp.float32(1.0))

        pltpu.sync_copy(fA0.at[pl.ds(0, PS)],
                        rc_ref.at[pl.ds(pl.multiple_of(b * R3 + off, 8), PS)])

    plsc.subcore_barrier()

    bufs = ((ibA, fA0, fA1, semA), (ibB, fB0, fB1, semB))

    def issue(ch, c0):
        ib, f0, f1, sem = bufs[ch % 2]
        n0 = ch * CH2
        h0 = pltpu.async_copy(
            idx_ref.at[pl.ds(pl.multiple_of(b * N + n0, 8), CH2)],
            ib.at[pl.ds(0, CH2)], sem)
        h1 = pltpu.async_copy(
            feat_ref.at[pl.ds(pl.multiple_of((b * C + c0) * N + n0, 8), CH2)],
            f0.at[pl.ds(0, CH2)], sem)
        h2 = pltpu.async_copy(
            feat_ref.at[pl.ds(pl.multiple_of((b * C + c0 + 1) * N + n0, 8), CH2)],
            f1.at[pl.ds(0, CH2)], sem)
        return (h0, h1, h2)

    out_handles = []
    for p in range(4):
        c0 = p * 8 + q * 2

        pending = issue(0, c0)

        for h in out_handles:
            h.wait()
        out_handles = []

        @plsc.parallel_loop(0, R3 // L, 1, unroll=8)
        def zero2(i):
            o = pl.multiple_of(i * L, L)
            bins0[pl.ds(o, L)] = zeros
            bins1[pl.ds(o, L)] = zeros

        for ch in range(NCH2):
            nxt = issue(ch + 1, c0) if ch + 1 < NCH2 else None
            for h in pending:
                h.wait()
            ib, f0, f1, _ = bufs[ch % 2]

            @plsc.parallel_loop(0, G2, 1, unroll=U)
            def p2_group(g):
                o = pl.multiple_of(g * L, L)
                iv = ib[pl.ds(o, L)]
                plsc.addupdate_scatter(bins0, [iv], f0[pl.ds(o, L)])
                plsc.addupdate_scatter(bins1, [iv], f1[pl.ds(o, L)])
            pending = nxt

        for sl in range(R3 // SUB):
            off = sl * SUB
            pltpu.sync_copy(
                rc_ref.at[pl.ds(pl.multiple_of(b * R3 + off, 8), SUB)],
                fA0.at[pl.ds(0, SUB)])

            @plsc.parallel_loop(0, SUB // L, 1, unroll=4)
            def scale(g):
                o = pl.multiple_of(g * L, L)
                r = fA0[pl.ds(o, L)]
                bins0[pl.ds(off + o, L)] = bins0[pl.ds(off + o, L)] * r
                bins1[pl.ds(off + o, L)] = bins1[pl.ds(off + o, L)] * r

            obase = (b * C + c0) * R3 + off
            out_handles.append(pltpu.async_copy(
                bins0.at[pl.ds(off, SUB)],
                out_ref.at[pl.ds(pl.multiple_of(obase, 8), SUB)], semB))
            out_handles.append(pltpu.async_copy(
                bins1.at[pl.ds(off, SUB)],
                out_ref.at[pl.ds(pl.multiple_of(obase + R3, 8), SUB)], semB))

    for h in out_handles:
        h.wait()


@jax.jit
def _grid_encode(pts_flat, feat_flat):
    mesh = plsc.VectorSubcoreMesh(core_axis_name="c", subcore_axis_name="s")
    fn = pl.kernel(
        _body,
        out_type=(
            jax.ShapeDtypeStruct((B * C * R3,), jnp.float32),
            jax.ShapeDtypeStruct((B * N,), jnp.int32),
            jax.ShapeDtypeStruct((32 * R3,), jnp.float32),
            jax.ShapeDtypeStruct((B * R3,), jnp.float32),
        ),
        mesh=mesh,
        compiler_params=pltpu.CompilerParams(needs_layout_passes=False),
        scratch_types=[
            pltpu.VMEM((R3,), jnp.float32),
            pltpu.VMEM((R3,), jnp.float32),
            pltpu.VMEM((CH2,), jnp.int32),
            pltpu.VMEM((CH2,), jnp.int32),
            pltpu.VMEM((CH2,), jnp.float32),
            pltpu.VMEM((CH2,), jnp.float32),
            pltpu.VMEM((CH2,), jnp.float32),
            pltpu.VMEM((CH2,), jnp.float32),
            pltpu.SemaphoreType.DMA,
            pltpu.SemaphoreType.DMA,
        ],
    )
    return fn(pts_flat, feat_flat)


def kernel(points, feature):
    pts_flat = points.transpose(0, 2, 1).reshape(-1)
    feat_flat = feature.reshape(-1)
    grid, _, _, _ = _grid_encode(pts_flat, feat_flat)
    return grid.reshape(B, C, R, R, R)


# speedup vs baseline: 6.3984x; 1.0465x over previous
import jax
import jax.numpy as jnp
from jax import lax
from jax.experimental import pallas as pl
from jax.experimental.pallas import tpu as pltpu
from jax.experimental.pallas import tpu_sc as plsc

B = 8
C = 32
N = 100000
R = 32
R3 = R * R * R
L = 16

CH1 = 2000
G1 = CH1 // L
CH2 = 10000
G2 = CH2 // L
NCH2 = N // CH2
U = 5
SUB = 8192
PS = 2048

CLIP_HI = 1.0 - 1e-6
RANGE = 2.2


def _voxel_index(xv, yv, zv):
    def quant(v):
        c = jnp.clip(v / jnp.float32(RANGE) + jnp.float32(0.5),
                     jnp.float32(0.0), jnp.float32(CLIP_HI))
        g = (c * jnp.float32(R)).astype(jnp.int32)
        return jnp.clip(g, 0, R - 1)
    return quant(xv) + R * quant(yv) + (R * R) * quant(zv)


def _tile_coords():
    cid = lax.axis_index("c")
    sid = lax.axis_index("s")
    w = cid * 16 + sid
    return w, w // 4, w % 4


def _points_body(pts_ref, idx_ref, rc_ref, pc_ref, cnt, ib1, s0, s1, s2, s3):
    w, b, q = _tile_coords()
    ones = jnp.ones((L,), jnp.float32)
    zeros = jnp.zeros((L,), jnp.float32)

    @plsc.parallel_loop(0, R3 // L, 1, unroll=8)
    def zero_body(i):
        cnt[pl.ds(pl.multiple_of(i * L, L), L)] = zeros

    nch = jnp.where(q < 2, 13, 12)
    ch_start = q * 13 - jnp.maximum(q - 2, 0)

    def p1_chunk(i, _):
        n0 = (ch_start + i) * CH1
        pbase = b * (3 * N) + n0
        pltpu.sync_copy(pts_ref.at[pl.ds(pl.multiple_of(pbase, 8), CH1)],
                        s0.at[pl.ds(0, CH1)])
        pltpu.sync_copy(pts_ref.at[pl.ds(pl.multiple_of(pbase + N, 8), CH1)],
                        s1.at[pl.ds(0, CH1)])
        pltpu.sync_copy(pts_ref.at[pl.ds(pl.multiple_of(pbase + 2 * N, 8), CH1)],
                        s2.at[pl.ds(0, CH1)])

        @plsc.parallel_loop(0, G1, 1, unroll=4)
        def p1_group(g):
            o = pl.multiple_of(g * L, L)
            iv = _voxel_index(s0[pl.ds(o, L)], s1[pl.ds(o, L)],
                              s2[pl.ds(o, L)])
            ib1[pl.ds(o, L)] = iv
            plsc.addupdate_scatter(cnt, [iv], ones)

        dst = b * N + n0
        pltpu.sync_copy(ib1.at[pl.ds(0, CH1)],
                        idx_ref.at[pl.ds(pl.multiple_of(dst, 8), CH1)])
        return 0
    lax.fori_loop(0, nch, p1_chunk, 0)

    pltpu.sync_copy(cnt.at[pl.ds(0, R3)],
                    pc_ref.at[pl.ds(pl.multiple_of(w * R3, 8), R3)])

    plsc.subcore_barrier()

    for sub in range(4):
        off = q * (R3 // 4) + sub * PS
        base = 4 * b * R3 + off
        pltpu.sync_copy(pc_ref.at[pl.ds(pl.multiple_of(base, 8), PS)],
                        s0.at[pl.ds(0, PS)])
        pltpu.sync_copy(pc_ref.at[pl.ds(pl.multiple_of(base + R3, 8), PS)],
                        s1.at[pl.ds(0, PS)])
        pltpu.sync_copy(pc_ref.at[pl.ds(pl.multiple_of(base + 2 * R3, 8), PS)],
                        s2.at[pl.ds(0, PS)])
        pltpu.sync_copy(pc_ref.at[pl.ds(pl.multiple_of(base + 3 * R3, 8), PS)],
                        s3.at[pl.ds(0, PS)])

        @plsc.parallel_loop(0, PS // L, 1, unroll=4)
        def rc_group(g):
            o = pl.multiple_of(g * L, L)
            s = (s0[pl.ds(o, L)] + s1[pl.ds(o, L)]
                 + s2[pl.ds(o, L)] + s3[pl.ds(o, L)])
            s0[pl.ds(o, L)] = jnp.float32(1.0) / jnp.maximum(s, jnp.float32(1.0))

        pltpu.sync_copy(s0.at[pl.ds(0, PS)],
                        rc_ref.at[pl.ds(pl.multiple_of(b * R3 + off, 8), PS)])


def _feat_body(feat_ref, idx_ref, rc_ref, out_ref,
               bins0, bins1, ibA, ibB, fA0, fA1, fB0, fB1, semA, semB):
    w, b, q = _tile_coords()
    zeros = jnp.zeros((L,), jnp.float32)

    bufs = ((ibA, fA0, fA1, semA), (ibB, fB0, fB1, semB))

    def issue(ch, c0):
        ib, f0, f1, sem = bufs[ch % 2]
        n0 = ch * CH2
        h0 = pltpu.async_copy(
            idx_ref.at[pl.ds(pl.multiple_of(b * N + n0, 8), CH2)],
            ib.at[pl.ds(0, CH2)], sem)
        h1 = pltpu.async_copy(
            feat_ref.at[pl.ds(pl.multiple_of((b * C + c0) * N + n0, 8), CH2)],
            f0.at[pl.ds(0, CH2)], sem)
        h2 = pltpu.async_copy(
            feat_ref.at[pl.ds(pl.multiple_of((b * C + c0 + 1) * N + n0, 8), CH2)],
            f1.at[pl.ds(0, CH2)], sem)
        return (h0, h1, h2)

    out_handles = []
    for p in range(4):
        c0 = p * 8 + q * 2

        pending = issue(0, c0)

        for h in out_handles:
            h.wait()
        out_handles = []

        @plsc.parallel_loop(0, R3 // L, 1, unroll=8)
        def zero2(i):
            o = pl.multiple_of(i * L, L)
            bins0[pl.ds(o, L)] = zeros
            bins1[pl.ds(o, L)] = zeros

        for ch in range(NCH2):
            nxt = issue(ch + 1, c0) if ch + 1 < NCH2 else None
            for h in pending:
                h.wait()
            ib, f0, f1, _ = bufs[ch % 2]

            @plsc.parallel_loop(0, G2, 1, unroll=U)
            def p2_group(g):
                o = pl.multiple_of(g * L, L)
                iv = ib[pl.ds(o, L)]
                plsc.addupdate_scatter(bins0, [iv], f0[pl.ds(o, L)])
                plsc.addupdate_scatter(bins1, [iv], f1[pl.ds(o, L)])
            pending = nxt

        for sl in range(R3 // SUB):
            off = sl * SUB
            pltpu.sync_copy(
                rc_ref.at[pl.ds(pl.multiple_of(b * R3 + off, 8), SUB)],
                fA0.at[pl.ds(0, SUB)])

            @plsc.parallel_loop(0, SUB // L, 1, unroll=4)
            def scale(g):
                o = pl.multiple_of(g * L, L)
                r = fA0[pl.ds(o, L)]
                bins0[pl.ds(off + o, L)] = bins0[pl.ds(off + o, L)] * r
                bins1[pl.ds(off + o, L)] = bins1[pl.ds(off + o, L)] * r

            obase = (b * C + c0) * R3 + off
            out_handles.append(pltpu.async_copy(
                bins0.at[pl.ds(off, SUB)],
                out_ref.at[pl.ds(pl.multiple_of(obase, 8), SUB)], semB))
            out_handles.append(pltpu.async_copy(
                bins1.at[pl.ds(off, SUB)],
                out_ref.at[pl.ds(pl.multiple_of(obase + R3, 8), SUB)], semB))

    for h in out_handles:
        h.wait()


@jax.jit
def _grid_encode(pts_flat, feat_flat):
    mesh = plsc.VectorSubcoreMesh(core_axis_name="c", subcore_axis_name="s")
    points_call = pl.kernel(
        _points_body,
        out_type=(
            jax.ShapeDtypeStruct((B * N,), jnp.int32),
            jax.ShapeDtypeStruct((B * R3,), jnp.float32),
            jax.ShapeDtypeStruct((32 * R3,), jnp.float32),
        ),
        mesh=mesh,
        compiler_params=pltpu.CompilerParams(needs_layout_passes=False),
        scratch_types=[
            pltpu.VMEM((R3,), jnp.float32),
            pltpu.VMEM((CH1,), jnp.int32),
            pltpu.VMEM((PS,), jnp.float32),
            pltpu.VMEM((PS,), jnp.float32),
            pltpu.VMEM((PS,), jnp.float32),
            pltpu.VMEM((PS,), jnp.float32),
        ],
    )
    idx, rc, _ = points_call(pts_flat)

    feat_call = pl.kernel(
        _feat_body,
        out_type=jax.ShapeDtypeStruct((B * C * R3,), jnp.float32),
        mesh=mesh,
        compiler_params=pltpu.CompilerParams(needs_layout_passes=False),
        scratch_types=[
            pltpu.VMEM((R3,), jnp.float32),
            pltpu.VMEM((R3,), jnp.float32),
            pltpu.VMEM((CH2,), jnp.int32),
            pltpu.VMEM((CH2,), jnp.int32),
            pltpu.VMEM((CH2,), jnp.float32),
            pltpu.VMEM((CH2,), jnp.float32),
            pltpu.VMEM((CH2,), jnp.float32),
            pltpu.VMEM((CH2,), jnp.float32),
            pltpu.SemaphoreType.DMA,
            pltpu.SemaphoreType.DMA,
        ],
    )
    return feat_call(feat_flat, idx, rc)


def kernel(points, feature):
    pts_flat = points.transpose(0, 2, 1).reshape(-1)
    feat_flat = feature.reshape(-1)
    grid = _grid_encode(pts_flat, feat_flat)
    return grid.reshape(B, C, R, R, R)
